# baseline (speedup 1.0000x reference)
import functools

import jax
import jax.numpy as jnp
from jax import lax
from jax.experimental import pallas as pl
from jax.experimental.pallas import tpu as pltpu
from jax.experimental.pallas import tpu_sc as plsc

FEAT = 64
BATCH = 16384
HIST = 50
NUM_EMB = 1000000

NUM_CORES = 2
NUM_SUBCORES = 16
NW = NUM_CORES * NUM_SUBCORES

_MESH = plsc.VectorSubcoreMesh(core_axis_name="c", subcore_axis_name="s")
_PARAMS = pltpu.CompilerParams(use_tc_tiling_on_sc=True, needs_layout_passes=False)


NBLK = 3906


@functools.partial(
    pl.kernel,
    mesh=_MESH,
    out_type=jax.ShapeDtypeStruct((NUM_EMB // 2, 128), jnp.float32),
    scratch_types=[
        pltpu.VMEM((2, 64, 256), jnp.float32),
        pltpu.VMEM((2, 128, 128), jnp.float32),
        pltpu.VMEM((32, 128), jnp.float32),
        pltpu.SemaphoreType.DMA((2,)),
        pltpu.SemaphoreType.DMA((2,)),
    ],
    compiler_params=_PARAMS,
)
def _transpose_table(emb_t, tail_t2, t2, inb, outb, tailb, sem_in, sem_out):
    w = lax.axis_index("s") * NUM_CORES + lax.axis_index("c")
    iota = lax.broadcasted_iota(jnp.int32, (16,), 0)
    rows_g = [(g * 16 + iota) // 2 for g in range(16)]
    colb_g = [((g * 16 + iota) % 2) * 64 for g in range(16)]

    def read_blk(c, buf):
        return pltpu.make_async_copy(
            emb_t.at[:, pl.ds(c * 256, 256)], inb.at[buf], sem_in.at[buf]
        )

    def write_blk(c, buf):
        return pltpu.make_async_copy(
            outb.at[buf], t2.at[pl.ds(c * 128, 128)], sem_out.at[buf]
        )

    def transpose_blk(buf):
        @plsc.parallel_loop(0, 64, unroll=16)
        def per_f(f):
            for g in range(16):
                v = inb[buf, f, pl.ds(g * 16, 16)]
                plsc.store_scatter(
                    outb.at[buf], [rows_g[g], colb_g[g] + f], v
                )

    @pl.when(w < NBLK)
    def _prologue():
        read_blk(w, 0).start()

    def body(t, carry):
        for par in range(2):
            nb = 2 * t + par
            c = nb * 32 + w
            nxt = c + 32

            @pl.when(nxt < NBLK)
            def _prefetch(par=par, nxt=nxt):
                read_blk(nxt, 1 - par).start()

            @pl.when(c < NBLK)
            def _work(par=par, c=c, nb=nb):
                read_blk(c, par).wait()

                @pl.when(nb >= 2)
                def _drain(par=par, c=c):
                    write_blk(c - 64, par).wait()

                transpose_blk(par)
                write_blk(c, par).start()
        return carry

    lax.fori_loop(0, 62, body, 0)

    nlast = (NBLK - 1 - w) // 32
    for par in range(2):
        nb_par = jnp.where(lax.rem(nlast, 2) == par, nlast, nlast - 1)
        @pl.when(nb_par >= 0)
        def _final(par=par, nb_par=nb_par):
            write_blk(nb_par * 32 + w, par).wait()

    @pl.when(w == 0)
    def _tail():
        pltpu.sync_copy(tail_t2, tailb)
        pltpu.sync_copy(tailb, t2.at[pl.ds(NBLK * 128, 32)])


UNITS_PER_W = (HIST * (BATCH // 256)) // NW


@functools.partial(
    pl.kernel,
    mesh=_MESH,
    out_type=jax.ShapeDtypeStruct((HIST, FEAT, BATCH), jnp.float32),
    scratch_types=[
        pltpu.VMEM((2, 256), jnp.int32),
        pltpu.VMEM((2, 2, 128), jnp.int32),
        pltpu.VMEM((2, 256), jnp.int32),
        pltpu.VMEM((2, 256, 128), jnp.float32),
        pltpu.VMEM((2, 64, 256), jnp.float32),
        pltpu.SemaphoreType.DMA((2,)),
        pltpu.SemaphoreType.DMA((2,)),
    ],
    compiler_params=_PARAMS,
)
def _gather_rows(idx_t, t2, out, idxv, idxp, half, g_buf, o_buf, sem_g, sem_w):
    w = lax.axis_index("s") * NUM_CORES + lax.axis_index("c")
    base_u = w * UNITS_PER_W
    iota = lax.broadcasted_iota(jnp.int32, (16,), 0)
    rows_g = [iota + g * 16 for g in range(16)]

    def unit_hc(u):
        return u // 64, lax.rem(u, 64)

    def load_indices(u, buf):
        h, cb = unit_hc(u)
        pltpu.sync_copy(idx_t.at[h, pl.ds(cb * 256, 256)], idxv.at[buf])
        for g in range(16):
            v = idxv[buf, pl.ds(g * 16, 16)]
            idxp[buf, g // 8, pl.ds((g % 8) * 16, 16)] = (
                lax.shift_right_logical(v, 1)
            )
            half[buf, pl.ds(g * 16, 16)] = lax.shift_left(
                jnp.bitwise_and(v, 1), 6
            )

    def start_gather(buf, k):
        return pltpu.make_async_copy(
            t2.at[idxp.at[buf, k]],
            g_buf.at[buf].at[pl.ds(k * 128, 128)],
            sem_g.at[buf],
        )

    def write_unit(u, buf):
        h, cb = unit_hc(u)
        return pltpu.make_async_copy(
            o_buf.at[buf], out.at[h, :, pl.ds(cb * 256, 256)], sem_w.at[buf]
        )

    def compact(buf):
        halves = [half[buf, pl.ds(g * 16, 16)] for g in range(16)]

        @plsc.parallel_loop(0, 64, unroll=16)
        def per_f(f):
            for g in range(16):
                val = plsc.load_gather(
                    g_buf.at[buf], [rows_g[g], halves[g] + f]
                )
                o_buf[buf, f, pl.ds(g * 16, 16)] = val

    load_indices(base_u, 0)
    start_gather(0, 0).start()
    start_gather(0, 1).start()

    def body(t, carry):
        for par in range(2):
            j = 2 * t + par
            u = base_u + j

            @pl.when(j + 1 < UNITS_PER_W)
            def _next(par=par, u=u):
                load_indices(u + 1, 1 - par)
                start_gather(1 - par, 0).start()
                start_gather(1 - par, 1).start()

            start_gather(par, 0).wait()
            start_gather(par, 1).wait()

            @pl.when(j >= 2)
            def _drain(par=par, u=u):
                write_unit(u - 2, par).wait()

            compact(par)
            write_unit(u, par).start()
        return carry

    lax.fori_loop(0, UNITS_PER_W // 2, body, 0)

    for par in range(2):
        write_unit(base_u + UNITS_PER_W - 2 + par, par).wait()


def kernel(inputs, embedding):
    tail_t2 = embedding[NBLK * 256:].reshape(32, 128)
    t2 = _transpose_table(embedding.T, tail_t2)
    o = _gather_rows(inputs.T, t2)
    return o.transpose(2, 0, 1)


# speedup vs baseline: 1.3461x; 1.3461x over previous
import functools

import jax
import jax.numpy as jnp
from jax import lax
from jax.experimental import pallas as pl
from jax.experimental.pallas import tpu as pltpu
from jax.experimental.pallas import tpu_sc as plsc

FEAT = 64
NUM_EMB = 1000000
BATCH = 16384
HIST = 50
TOTAL = BATCH * HIST

NUM_CORES = 2
NUM_SUBCORES = 16
NUM_WORKERS = NUM_CORES * NUM_SUBCORES
B_PER_W = TOTAL // NUM_WORKERS

CHUNK = 256
NCHUNKS = B_PER_W // CHUNK
NBUF = 4
NGROUPS = NCHUNKS // NBUF

_MESH = plsc.VectorSubcoreMesh(core_axis_name="c", subcore_axis_name="s")


@functools.partial(
    pl.kernel,
    mesh=_MESH,
    out_type=jax.ShapeDtypeStruct((TOTAL, FEAT), jnp.float32),
    scratch_types=[
        pltpu.VMEM((NCHUNKS, CHUNK), jnp.int32),
        pltpu.VMEM((NBUF, CHUNK, FEAT), jnp.float32),
        pltpu.SemaphoreType.DMA((NBUF,)),
        pltpu.SemaphoreType.DMA((NBUF,)),
    ],
    compiler_params=pltpu.CompilerParams(use_tc_tiling_on_sc=False),
)
def _gather_all_tiles(idx_hbm, table_hbm, out_hbm, idx_v, rows_v, sem_g, sem_s):
    wid = lax.axis_index("s") * NUM_CORES + lax.axis_index("c")
    base = wid * B_PER_W

    pltpu.sync_copy(idx_hbm.at[wid], idx_v)

    def group(g, carry):
        for b in range(NBUF):
            i = g * NBUF + b

            @pl.when(g > 0)
            def _wait_prev_store(b=b, i=i):
                prev_off = base + (i - NBUF) * CHUNK
                pltpu.make_async_copy(
                    rows_v.at[b], out_hbm.at[pl.ds(prev_off, CHUNK)], sem_s.at[b]
                ).wait()

            pltpu.make_async_copy(
                table_hbm.at[idx_v.at[i]], rows_v.at[b], sem_g.at[b]
            ).start()

        for b in range(NBUF):
            i = g * NBUF + b
            pltpu.make_async_copy(
                table_hbm.at[idx_v.at[i]], rows_v.at[b], sem_g.at[b]
            ).wait()
            pltpu.make_async_copy(
                rows_v.at[b], out_hbm.at[pl.ds(base + i * CHUNK, CHUNK)], sem_s.at[b]
            ).start()
        return carry

    lax.fori_loop(0, NGROUPS, group, 0)

    for b in range(NBUF):
        i = (NGROUPS - 1) * NBUF + b
        pltpu.make_async_copy(
            rows_v.at[b], out_hbm.at[pl.ds(base + i * CHUNK, CHUNK)], sem_s.at[b]
        ).wait()


def kernel(inputs, embedding):
    idx = (inputs * 2).reshape(NUM_WORKERS, NCHUNKS, CHUNK).astype(jnp.int32)
    table = jnp.pad(embedding, ((0, 0), (0, FEAT))).reshape(2 * NUM_EMB, FEAT)
    out = _gather_all_tiles(idx, table)
    return out.reshape(inputs.shape + (FEAT,))
